# Initial kernel scaffold; baseline (speedup 1.0000x reference)
import jax
import jax.numpy as jnp
from jax.experimental import pallas as pl


def kernel(x, edge_index, W1, b1, W2, b2, W3, b3):
    raise NotImplementedError("write your pallas kernel here")


# speedup vs baseline: 6.5224x; 6.5224x over previous
import functools

import jax
import jax.numpy as jnp
from jax import lax
from jax.experimental import pallas as pl
from jax.experimental.pallas import tpu as pltpu
from jax.experimental.pallas import tpu_sc as plsc

N = 10000
NP = 10240
D = 256
E = 160000
HALF = D // 2

NC = 2
NS = 16
B = 128
NCH = 79
EP = NS * NCH * B
RPT = NP // NS

_MESH = plsc.VectorSubcoreMesh(core_axis_name="c", subcore_axis_name="s")


@functools.partial(
    pl.kernel,
    mesh=_MESH,
    out_type=(
        jax.ShapeDtypeStruct((NP, 16), jnp.float32),
        jax.ShapeDtypeStruct((NP, 16), jnp.float32),
    ),
    scratch_types=[
        pltpu.VMEM((NCH, B), jnp.int32),
        pltpu.VMEM((B, 16), jnp.float32),
        pltpu.VMEM_SHARED((NP, 16), jnp.float32),
    ],
)
def _deg_sc(dst_hbm, ones_hbm, zeros_hbm, degA_hbm, degB_hbm, dst_v, ones_v, acc):
    cid = lax.axis_index("c")
    sid = lax.axis_index("s")
    rows = pl.ds(sid * RPT, RPT)
    pltpu.sync_copy(dst_hbm.at[sid], dst_v)
    pltpu.sync_copy(ones_hbm, ones_v)
    pltpu.sync_copy(zeros_hbm, acc.at[rows])
    plsc.subcore_barrier()

    def body(j, carry):
        pltpu.sync_copy(ones_v, acc.at[dst_v.at[j]], add=True)
        return carry

    @pl.when(cid == 0)
    def _():
        lax.fori_loop(0, NCH // 2, body, 0)

    @pl.when(cid == 1)
    def _():
        lax.fori_loop(NCH // 2, NCH, body, 0)

    plsc.subcore_barrier()

    @pl.when(cid == 0)
    def _():
        pltpu.sync_copy(acc.at[rows], degA_hbm.at[rows])

    @pl.when(cid == 1)
    def _():
        pltpu.sync_copy(acc.at[rows], degB_hbm.at[rows])


@functools.partial(
    pl.kernel,
    mesh=_MESH,
    out_type=(
        jax.ShapeDtypeStruct((NP, HALF), jnp.float32),
        jax.ShapeDtypeStruct((NP, HALF), jnp.float32),
    ),
    scratch_types=[
        pltpu.VMEM((NCH, B), jnp.int32),
        pltpu.VMEM((NCH, B), jnp.int32),
        pltpu.VMEM((B, HALF), jnp.float32),
        pltpu.VMEM_SHARED((NP, HALF), jnp.float32),
        pltpu.SemaphoreType.DMA,
    ],
)
def _agg_sc(gL_hbm, gR_hbm, src_hbm, dst_hbm, outL_hbm, outR_hbm,
            src_v, dst_v, rows_v, acc, sem):
    cid = lax.axis_index("c")
    sid = lax.axis_index("s")
    rows = pl.ds(sid * RPT, RPT)
    pltpu.sync_copy(src_hbm.at[sid], src_v)
    pltpu.sync_copy(dst_hbm.at[sid], dst_v)

    @pl.when(cid == 0)
    def _():
        pltpu.sync_copy(gL_hbm.at[rows], acc.at[rows])

    @pl.when(cid == 1)
    def _():
        pltpu.sync_copy(gR_hbm.at[rows], acc.at[rows])

    plsc.subcore_barrier()

    def body(j, carry):
        @pl.when(cid == 0)
        def _():
            pltpu.async_copy(gL_hbm.at[src_v.at[j]], rows_v, sem).wait()

        @pl.when(cid == 1)
        def _():
            pltpu.async_copy(gR_hbm.at[src_v.at[j]], rows_v, sem).wait()

        pltpu.sync_copy(rows_v, acc.at[dst_v.at[j]], add=True)
        return carry

    lax.fori_loop(0, NCH, body, 0)
    plsc.subcore_barrier()

    @pl.when(cid == 0)
    def _():
        pltpu.sync_copy(acc.at[rows], outL_hbm.at[rows])

    @pl.when(cid == 1)
    def _():
        pltpu.sync_copy(acc.at[rows], outR_hbm.at[rows])


_R = 1000


def _dinv_of(degA_ref, degB_ref):
    return lax.rsqrt(degA_ref[:, :1] + degB_ref[:, :1] + 1.0)


def _mm1_body(x_ref, w_ref, degA_ref, degB_ref, gL_ref, gR_ref):
    dinv = _dinv_of(degA_ref, degB_ref)
    g = jnp.dot(x_ref[...], w_ref[...], preferred_element_type=jnp.float32) * dinv
    gL_ref[...] = g[:, :HALF]
    gR_ref[...] = g[:, HALF:]


def _mid_body(aL_ref, aR_ref, degA_ref, degB_ref, b_ref, w_ref, gL_ref, gR_ref):
    dinv = _dinv_of(degA_ref, degB_ref)
    a = jnp.concatenate([aL_ref[...], aR_ref[...]], axis=1)
    h = jnp.maximum(a * dinv + b_ref[...], 0.0)
    g = jnp.dot(h, w_ref[...], preferred_element_type=jnp.float32) * dinv
    gL_ref[...] = g[:, :HALF]
    gR_ref[...] = g[:, HALF:]


def _fin_body(aL_ref, aR_ref, degA_ref, degB_ref, b_ref, out_ref):
    dinv = _dinv_of(degA_ref, degB_ref)
    a = jnp.concatenate([aL_ref[...], aR_ref[...]], axis=1)
    out_ref[...] = a * dinv + b_ref[...]


_row_spec = lambda w: pl.BlockSpec((_R, w), lambda i: (i, 0))
_full_spec = lambda r, c: pl.BlockSpec((r, c), lambda i: (0, 0))

_mm1 = pl.pallas_call(
    _mm1_body,
    grid=(N // _R,),
    in_specs=[_row_spec(D), _full_spec(D, D), _row_spec(16), _row_spec(16)],
    out_specs=[_row_spec(HALF), _row_spec(HALF)],
    out_shape=(
        jax.ShapeDtypeStruct((NP, HALF), jnp.float32),
        jax.ShapeDtypeStruct((NP, HALF), jnp.float32),
    ),
)

_mid = pl.pallas_call(
    _mid_body,
    grid=(N // _R,),
    in_specs=[_row_spec(HALF), _row_spec(HALF), _row_spec(16), _row_spec(16),
              _full_spec(1, D), _full_spec(D, D)],
    out_specs=[_row_spec(HALF), _row_spec(HALF)],
    out_shape=(
        jax.ShapeDtypeStruct((NP, HALF), jnp.float32),
        jax.ShapeDtypeStruct((NP, HALF), jnp.float32),
    ),
)

_fin = pl.pallas_call(
    _fin_body,
    grid=(N // _R,),
    in_specs=[_row_spec(HALF), _row_spec(HALF), _row_spec(16), _row_spec(16),
              _full_spec(1, D)],
    out_specs=_row_spec(D),
    out_shape=jax.ShapeDtypeStruct((N, D), jnp.float32),
)


def kernel(x, edge_index, W1, b1, W2, b2, W3, b3):
    src = jnp.concatenate(
        [edge_index[0].astype(jnp.int32), jnp.zeros((EP - E,), jnp.int32)]
    ).reshape(NS, NCH, B)
    dst = jnp.concatenate(
        [edge_index[1].astype(jnp.int32), jnp.full((EP - E,), N, jnp.int32)]
    ).reshape(NS, NCH, B)
    ones = jnp.ones((B, 16), jnp.float32)
    zeros = jnp.zeros((RPT, 16), jnp.float32)

    degA, degB = _deg_sc(dst, ones, zeros)
    gL, gR = _mm1(x, W1, degA, degB)
    aL, aR = _agg_sc(gL, gR, src, dst)
    gL, gR = _mid(aL, aR, degA, degB, b1.reshape(1, D), W2)
    aL, aR = _agg_sc(gL, gR, src, dst)
    gL, gR = _mid(aL, aR, degA, degB, b2.reshape(1, D), W3)
    aL, aR = _agg_sc(gL, gR, src, dst)
    return _fin(aL, aR, degA, degB, b3.reshape(1, D))
